# baseline (speedup 1.0000x reference)
import functools

import jax
import jax.numpy as jnp
from jax import lax
from jax.experimental import pallas as pl
from jax.experimental.pallas import tpu as pltpu
from jax.experimental.pallas import tpu_sc as plsc

BATCH = 16384
DIM = 64

_info = plsc.get_sparse_core_info()
_NC, _NS = _info.num_cores, _info.num_subcores
_NW = _NC * _NS
_BPW = BATCH // _NW
_CHUNK = 128
_NCHUNK = _BPW // _CHUNK


def _gather_body(ent_emb, ent_tr, rel_emb, rel_tr, ih, it, ir,
                 oh, ot, orr, oht, ott, ort,
                 ihv, itv, irv, rows_a, rows_b, sem_a, sem_b,
                 wsem_a, wsem_b):
    wid = lax.axis_index("s") * _NC + lax.axis_index("c")
    base = wid * _BPW

    pltpu.sync_copy(ih.at[pl.ds(base, _BPW)], ihv)
    pltpu.sync_copy(it.at[pl.ds(base, _BPW)], itv)
    pltpu.sync_copy(ir.at[pl.ds(base, _BPW)], irv)

    tasks = [
        (ihv, ent_emb, oh),
        (itv, ent_emb, ot),
        (ihv, ent_tr, oht),
        (itv, ent_tr, ott),
        (irv, rel_emb, orr),
        (irv, rel_tr, ort),
    ]
    bufs = [(rows_a, sem_a, wsem_a), (rows_b, sem_b, wsem_b)]
    pending_write = [None, None]
    prev = None
    for k, (iv, table, out) in enumerate(tasks):
        b = k % 2
        rows, gsem, wsem = bufs[b]
        if pending_write[b] is not None:
            pending_write[b].wait()
            pending_write[b] = None
        descs = []
        for j in range(_NCHUNK):
            descs.append(pltpu.async_copy(
                table.at[iv.at[pl.ds(j * _CHUNK, _CHUNK)]],
                rows.at[pl.ds(j * _CHUNK, _CHUNK)],
                gsem))
        if prev is not None:
            pdescs, prows, pout, pwsem, pb = prev
            for c in pdescs:
                c.wait()
            pending_write[pb] = pltpu.async_copy(
                prows, pout.at[pl.ds(base, _BPW)], pwsem)
        prev = (descs, rows, out, wsem, b)
    pdescs, prows, pout, pwsem, pb = prev
    for c in pdescs:
        c.wait()
    pending_write[pb] = pltpu.async_copy(
        prows, pout.at[pl.ds(base, _BPW)], pwsem)
    for w in pending_write:
        if w is not None:
            w.wait()


def _sc_gather(ent_emb, ent_tr, rel_emb, rel_tr, ih, it, ir):
    mesh = plsc.VectorSubcoreMesh(core_axis_name="c", subcore_axis_name="s")
    row_ty = jax.ShapeDtypeStruct((BATCH, DIM), jnp.float32)
    fn = pl.kernel(
        _gather_body,
        mesh=mesh,
        compiler_params=pltpu.CompilerParams(use_tc_tiling_on_sc=False),
        out_type=[row_ty] * 6,
        scratch_types=[
            pltpu.VMEM((_BPW,), jnp.int32),
            pltpu.VMEM((_BPW,), jnp.int32),
            pltpu.VMEM((_BPW,), jnp.int32),
            pltpu.VMEM((_BPW, DIM), jnp.float32),
            pltpu.VMEM((_BPW, DIM), jnp.float32),
            pltpu.SemaphoreType.DMA,
            pltpu.SemaphoreType.DMA,
            pltpu.SemaphoreType.DMA,
            pltpu.SemaphoreType.DMA,
        ],
    )
    return fn(ent_emb, ent_tr, rel_emb, rel_tr, ih, it, ir)


def _score_body(h_ref, t_ref, r_ref, ht_ref, tt_ref, rt_ref, o_ref):
    h = h_ref[...]
    t = t_ref[...]
    r = r_ref[...]
    ht = ht_ref[...]
    tt = tt_ref[...]
    rt = rt_ref[...]

    def _l2(x):
        n = jnp.sqrt(jnp.sum(x * x, axis=-1, keepdims=True))
        return x / jnp.maximum(n, 1e-12)

    ph = _l2(h + jnp.sum(h * ht, axis=-1, keepdims=True) * rt)
    pt = _l2(t + jnp.sum(t * tt, axis=-1, keepdims=True) * rt)
    ph = _l2(ph)
    pt = _l2(pt)
    rn = _l2(r)
    o_ref[...] = jnp.sum(jnp.abs(ph + rn - pt), axis=-1)


def _tc_score(h, t, r, ht, tt, rt):
    blk = 2048
    grid = BATCH // blk
    in_spec = pl.BlockSpec((blk, DIM), lambda i: (i, 0))
    return pl.pallas_call(
        _score_body,
        grid=(grid,),
        in_specs=[in_spec] * 6,
        out_specs=pl.BlockSpec((blk,), lambda i: (i,)),
        out_shape=jax.ShapeDtypeStruct((BATCH,), jnp.float32),
    )(h, t, r, ht, tt, rt)


def kernel(batch_h, batch_t, batch_r, ent_embeddings, rel_embeddings,
           ent_transfer, rel_transfer):
    ih = batch_h.astype(jnp.int32)
    it = batch_t.astype(jnp.int32)
    ir = batch_r.astype(jnp.int32)
    h, t, r, ht, tt, rt = _sc_gather(
        ent_embeddings, ent_transfer, rel_embeddings, rel_transfer,
        ih, it, ir)
    return _tc_score(h, t, r, ht, tt, rt)
